# baseline (speedup 1.0000x reference)
import functools

import jax
import jax.numpy as jnp
from jax import lax
from jax.experimental import pallas as pl
from jax.experimental.pallas import tpu as pltpu
from jax.experimental.pallas import tpu_sc as plsc

_K = 2048
_B = 128
_N = 32768
_L = 16
_NVEC = _N // _L
_NB1 = 2048
_NB2 = 2048
_NB3 = 1024
_TIE_CAP = _K
_ROWS_PER_W = _B // 32


def _sc_mask(logits):
    mesh = plsc.VectorSubcoreMesh(core_axis_name="c", subcore_axis_name="s",
                                  num_cores=2, num_subcores=16)

    @functools.partial(
        pl.kernel,
        out_type=jax.ShapeDtypeStruct((_B, _N), jnp.float32),
        mesh=mesh,
        compiler_params=pltpu.CompilerParams(needs_layout_passes=False),
        scratch_types=[
            pltpu.VMEM((_N,), jnp.float32),
            pltpu.VMEM((_L * _NB1,), jnp.int32),
            pltpu.VMEM((_NB1 + _L,), jnp.int32),
            pltpu.VMEM((_TIE_CAP + 2 * _L,), jnp.int32),
        ],
    )
    def k(x_hbm, o_hbm, row_v, hist_v, suf_v, tie_v):
        wid = lax.axis_index("s") * 2 + lax.axis_index("c")
        lanes = jnp.arange(_L, dtype=jnp.int32)
        zeros_i = jnp.zeros((_L,), jnp.int32)
        ones_i = jnp.ones((_L,), jnp.int32)
        ones_f = jnp.ones((_L,), jnp.float32)
        zeros_f = jnp.zeros((_L,), jnp.float32)

        def zero_hist(nb):
            def z(i, _):
                hist_v[pl.ds(i * _L, _L)] = zeros_i
                return 0
            lax.fori_loop(0, (_L * nb) // _L, z, 0)

        def select(nb, k_rem):
            nchunk = nb // _L
            suf_v[pl.ds(nb, _L)] = zeros_i

            def tot(c, _):
                acc = hist_v[pl.ds(c * _L, _L)]
                for l in range(1, _L):
                    acc = acc + hist_v[pl.ds(l * nb + c * _L, _L)]
                suf_v[pl.ds(c * _L, _L)] = acc
                return 0
            lax.fori_loop(0, nchunk, tot, 0)

            def sfx(ci, carry):
                run, nge = carry
                c = nchunk - 1 - ci
                t = suf_v[pl.ds(c * _L, _L)]
                cs = plsc.cumsum(t)
                tot_c = jnp.sum(t)
                s_vec = (run + tot_c) - cs + t
                suf_v[pl.ds(c * _L, _L)] = s_vec
                nge = nge + jnp.sum((s_vec >= k_rem).astype(jnp.int32))
                return (run + tot_c, nge)
            _, nge = lax.fori_loop(0, nchunk, sfx,
                                   (jnp.int32(0), jnp.int32(0)))
            bstar = nge - 1
            s_next = plsc.load_gather(
                suf_v, [jnp.full((_L,), bstar + 1, jnp.int32)])
            r = k_rem - jnp.max(s_next)
            return bstar, r

        for j in range(_ROWS_PER_W):
            row = wid * _ROWS_PER_W + j
            pltpu.sync_copy(x_hbm.at[row], row_v)

            zero_hist(_NB1)

            def p1(i, _):
                v = row_v[pl.ds(i * _L, _L)]
                xi = lax.bitcast_convert_type(v, jnp.int32)
                sgn = lax.shift_right_arithmetic(xi, 31)
                uk = xi ^ (sgn | jnp.int32(-(2 ** 31)))
                row_v[pl.ds(i * _L, _L)] = lax.bitcast_convert_type(
                    uk, jnp.float32)
                b1 = lax.shift_right_logical(uk, 21)
                plsc.addupdate_scatter(hist_v, [lanes * _NB1 + b1], ones_i)
                return 0
            lax.fori_loop(0, _NVEC, p1, 0)
            b1s, k2 = select(_NB1, jnp.int32(_K))

            zero_hist(_NB2)
            b1v = jnp.full((_L,), b1s, jnp.int32)

            def p2(i, _):
                uk = lax.bitcast_convert_type(row_v[pl.ds(i * _L, _L)],
                                              jnp.int32)
                pm = lax.shift_right_logical(uk, 21) == b1v
                b2 = lax.shift_right_logical(uk, 10) & 0x7FF
                plsc.addupdate_scatter(hist_v, [lanes * _NB2 + b2], ones_i,
                                       mask=pm)
                return 0
            lax.fori_loop(0, _NVEC, p2, 0)
            b2s, k3 = select(_NB2, k2)

            zero_hist(_NB3)
            pfx22 = jnp.full((_L,), b1s * 2048 + b2s, jnp.int32)

            def p3(i, _):
                uk = lax.bitcast_convert_type(row_v[pl.ds(i * _L, _L)],
                                              jnp.int32)
                pm = lax.shift_right_logical(uk, 10) == pfx22
                b3 = uk & 0x3FF
                plsc.addupdate_scatter(hist_v, [lanes * _NB3 + b3], ones_i,
                                       mask=pm)
                return 0
            lax.fori_loop(0, _NVEC, p3, 0)
            b3s, r = select(_NB3, k3)

            tvec_u = lax.bitcast_convert_type(
                jnp.full((_L,), (lax.shift_left(b1s, 21)
                                 | lax.shift_left(b2s, 10) | b3s),
                         jnp.int32), jnp.uint32)

            def pm_loop(i, base_vec):
                uku = lax.bitcast_convert_type(row_v[pl.ds(i * _L, _L)],
                                               jnp.uint32)
                gt = uku > tvec_u
                eq = uku == tvec_u
                row_v[pl.ds(i * _L, _L)] = jnp.where(gt, ones_f, zeros_f)
                eq_i = eq.astype(jnp.int32)
                cs = plsc.cumsum(eq_i)
                pos = base_vec + cs - eq_i
                plsc.store_scatter(tie_v, [pos], lanes + i * _L, mask=eq)
                cnt = plsc.all_reduce_population_count(eq)
                return jnp.minimum(base_vec + cnt,
                                   jnp.full((_L,), _TIE_CAP, jnp.int32))
            lax.fori_loop(0, _NVEC, pm_loop, zeros_i)

            nfix = lax.div(r + (_L - 1), jnp.int32(_L))

            def fix(f, _):
                idxs = tie_v[pl.ds(f * _L, _L)]
                valid = lanes < (r - f * _L)
                plsc.store_scatter(row_v, [idxs], ones_f, mask=valid)
                return 0
            lax.fori_loop(0, nfix, fix, 0)

            pltpu.sync_copy(row_v, o_hbm.at[row])

    return k(logits)


def kernel(logits):
    return _sc_mask(logits) != 0


# speedup vs baseline: 1.2831x; 1.2831x over previous
import functools

import jax
import jax.numpy as jnp
from jax import lax
from jax.experimental import pallas as pl
from jax.experimental.pallas import tpu as pltpu
from jax.experimental.pallas import tpu_sc as plsc

_K = 2048
_B = 128
_N = 32768
_L = 16
_NVEC = _N // _L
_NB1 = 2048
_NB2 = 2048
_NB3 = 1024
_TIE_CAP = _K
_ROWS_PER_W = _B // 32


def _sc_mask(logits):
    mesh = plsc.VectorSubcoreMesh(core_axis_name="c", subcore_axis_name="s",
                                  num_cores=2, num_subcores=16)

    @functools.partial(
        pl.kernel,
        out_type=jax.ShapeDtypeStruct((_B, _N), jnp.float32),
        mesh=mesh,
        compiler_params=pltpu.CompilerParams(needs_layout_passes=False),
        scratch_types=[
            pltpu.VMEM((_N,), jnp.float32),
            pltpu.VMEM((_L * _NB1,), jnp.int32),
            pltpu.VMEM((_NB1 + _L,), jnp.int32),
            pltpu.VMEM((_TIE_CAP + 2 * _L,), jnp.int32),
        ],
    )
    def k(x_hbm, o_hbm, row_v, hist_v, suf_v, tie_v):
        wid = lax.axis_index("s") * 2 + lax.axis_index("c")
        lanes = jnp.arange(_L, dtype=jnp.int32)
        zeros_i = jnp.zeros((_L,), jnp.int32)
        ones_i = jnp.ones((_L,), jnp.int32)
        ones_f = jnp.ones((_L,), jnp.float32)
        zeros_f = jnp.zeros((_L,), jnp.float32)

        def zero_hist(nb):
            def z(i, _):
                hist_v[pl.ds(i * _L, _L)] = zeros_i
                return 0
            lax.fori_loop(0, (_L * nb) // _L, z, 0, unroll=8)

        def select(nb, k_rem):
            nchunk = nb // _L
            suf_v[pl.ds(nb, _L)] = zeros_i

            def tot(c, _):
                acc = hist_v[pl.ds(c * _L, _L)]
                for l in range(1, _L):
                    acc = acc + hist_v[pl.ds(l * nb + c * _L, _L)]
                suf_v[pl.ds(c * _L, _L)] = acc
                return 0
            lax.fori_loop(0, nchunk, tot, 0, unroll=2)

            def sfx(ci, carry):
                run, nge = carry
                c = nchunk - 1 - ci
                t = suf_v[pl.ds(c * _L, _L)]
                cs = plsc.cumsum(t)
                tot_c = jnp.sum(t)
                s_vec = (run + tot_c) - cs + t
                suf_v[pl.ds(c * _L, _L)] = s_vec
                nge = nge + jnp.sum((s_vec >= k_rem).astype(jnp.int32))
                return (run + tot_c, nge)
            _, nge = lax.fori_loop(0, nchunk, sfx,
                                   (jnp.int32(0), jnp.int32(0)))
            bstar = nge - 1
            s_next = plsc.load_gather(
                suf_v, [jnp.full((_L,), bstar + 1, jnp.int32)])
            r = k_rem - jnp.max(s_next)
            return bstar, r

        for j in range(_ROWS_PER_W):
            row = wid * _ROWS_PER_W + j
            pltpu.sync_copy(x_hbm.at[row], row_v)

            zero_hist(_NB1)

            def p1(i, _):
                v = row_v[pl.ds(i * _L, _L)]
                xi = lax.bitcast_convert_type(v, jnp.int32)
                sgn = lax.shift_right_arithmetic(xi, 31)
                uk = xi ^ (sgn | jnp.int32(-(2 ** 31)))
                row_v[pl.ds(i * _L, _L)] = lax.bitcast_convert_type(
                    uk, jnp.float32)
                b1 = lax.shift_right_logical(uk, 21)
                plsc.addupdate_scatter(hist_v, [lanes * _NB1 + b1], ones_i)
                return 0
            lax.fori_loop(0, _NVEC, p1, 0, unroll=8)
            b1s, k2 = select(_NB1, jnp.int32(_K))

            zero_hist(_NB2)
            b1v = jnp.full((_L,), b1s, jnp.int32)

            def p2(i, _):
                uk = lax.bitcast_convert_type(row_v[pl.ds(i * _L, _L)],
                                              jnp.int32)
                pm = lax.shift_right_logical(uk, 21) == b1v
                b2 = lax.shift_right_logical(uk, 10) & 0x7FF
                plsc.addupdate_scatter(hist_v, [lanes * _NB2 + b2], ones_i,
                                       mask=pm)
                return 0
            lax.fori_loop(0, _NVEC, p2, 0, unroll=8)
            b2s, k3 = select(_NB2, k2)

            zero_hist(_NB3)
            pfx22 = jnp.full((_L,), b1s * 2048 + b2s, jnp.int32)

            def p3(i, _):
                uk = lax.bitcast_convert_type(row_v[pl.ds(i * _L, _L)],
                                              jnp.int32)
                pm = lax.shift_right_logical(uk, 10) == pfx22
                b3 = uk & 0x3FF
                plsc.addupdate_scatter(hist_v, [lanes * _NB3 + b3], ones_i,
                                       mask=pm)
                return 0
            lax.fori_loop(0, _NVEC, p3, 0, unroll=8)
            b3s, r = select(_NB3, k3)

            tvec_u = lax.bitcast_convert_type(
                jnp.full((_L,), (lax.shift_left(b1s, 21)
                                 | lax.shift_left(b2s, 10) | b3s),
                         jnp.int32), jnp.uint32)

            def pm_loop(i, base_vec):
                uku = lax.bitcast_convert_type(row_v[pl.ds(i * _L, _L)],
                                               jnp.uint32)
                gt = uku > tvec_u
                eq = uku == tvec_u
                row_v[pl.ds(i * _L, _L)] = jnp.where(gt, ones_f, zeros_f)
                eq_i = eq.astype(jnp.int32)
                cs = plsc.cumsum(eq_i)
                pos = base_vec + cs - eq_i
                plsc.store_scatter(tie_v, [pos], lanes + i * _L, mask=eq)
                cnt = plsc.all_reduce_population_count(eq)
                return jnp.minimum(base_vec + cnt,
                                   jnp.full((_L,), _TIE_CAP, jnp.int32))
            lax.fori_loop(0, _NVEC, pm_loop, zeros_i, unroll=4)

            nfix = lax.div(r + (_L - 1), jnp.int32(_L))

            def fix(f, _):
                idxs = tie_v[pl.ds(f * _L, _L)]
                valid = lanes < (r - f * _L)
                plsc.store_scatter(row_v, [idxs], ones_f, mask=valid)
                return 0
            lax.fori_loop(0, nfix, fix, 0)

            pltpu.sync_copy(row_v, o_hbm.at[row])

    return k(logits)


def kernel(logits):
    return _sc_mask(logits) != 0
